# Initial kernel scaffold; baseline (speedup 1.0000x reference)
import jax
import jax.numpy as jnp
from jax.experimental import pallas as pl


def kernel(coord, charges, nbmat_lr, mol_idx):
    raise NotImplementedError("write your pallas kernel here")


# speedup vs baseline: 38.9716x; 38.9716x over previous
import functools

import jax
import jax.numpy as jnp
from jax import lax
from jax.experimental import pallas as pl
from jax.experimental.pallas import tpu as pltpu
from jax.experimental.pallas import tpu_sc as plsc

N = 100000
M = 64
NMOL = 256
RC = 4.6
FACTOR = 13.605693122994 * 0.529177210903

NC = 2
NS = 16
NW = NC * NS
L = 16

CHUNK = 16
PER_TILE = 3136
NPAD = NW * PER_TILE
NBLK = PER_TILE // CHUNK
PAIRS = CHUNK * M

_mesh = plsc.VectorSubcoreMesh(
    core_axis_name="c", subcore_axis_name="s", num_cores=NC, num_subcores=NS
)


@functools.partial(
    pl.kernel,
    out_type=jax.ShapeDtypeStruct((NW, NMOL), jnp.float32),
    mesh=_mesh,
    compiler_params=pltpu.CompilerParams(needs_layout_passes=False),
    scratch_types=[
        pltpu.VMEM((PAIRS,), jnp.int32),
        pltpu.VMEM((PAIRS,), jnp.float32),
        pltpu.VMEM((PAIRS,), jnp.float32),
        pltpu.VMEM((PAIRS,), jnp.float32),
        pltpu.VMEM((PAIRS,), jnp.float32),
        pltpu.VMEM((PER_TILE,), jnp.float32),
        pltpu.VMEM((PER_TILE,), jnp.float32),
        pltpu.VMEM((PER_TILE,), jnp.float32),
        pltpu.VMEM((PER_TILE,), jnp.float32),
        pltpu.VMEM((PER_TILE,), jnp.int32),
        pltpu.VMEM((L * NMOL,), jnp.float32),
        pltpu.VMEM((NMOL,), jnp.float32),
        pltpu.SemaphoreType.DMA,
        pltpu.SemaphoreType.DMA,
    ],
)
def _lr_coulomb_sc(xt_hbm, yt_hbm, zt_hbm, qt_hbm, nb_hbm, mol_hbm, out_hbm,
                   idx_v, xr, yr, zr, qr, ox, oy, oz, oq, omol,
                   molacc, res_v, gsem, ssem):
    wid = lax.axis_index("s") * NC + lax.axis_index("c")
    start = wid * PER_TILE

    iota = lax.broadcasted_iota(jnp.int32, (L,), 0)
    zf = jnp.zeros((L,), jnp.float32)

    own_cps = [
        pltpu.async_copy(xt_hbm.at[pl.ds(start, PER_TILE)], ox, ssem),
        pltpu.async_copy(yt_hbm.at[pl.ds(start, PER_TILE)], oy, ssem),
        pltpu.async_copy(zt_hbm.at[pl.ds(start, PER_TILE)], oz, ssem),
        pltpu.async_copy(qt_hbm.at[pl.ds(start, PER_TILE)], oq, ssem),
        pltpu.async_copy(mol_hbm.at[pl.ds(start, PER_TILE)], omol, ssem),
    ]
    for cp in own_cps:
        cp.wait()

    def zero_body(i, _):
        molacc[pl.ds(i * L, L)] = zf
        return 0
    lax.fori_loop(0, (L * NMOL) // L, zero_body, 0)

    inv_rc2 = jnp.float32(1.0 / (RC * RC))

    def block_body(b, _):
        base = start + b * CHUNK
        pltpu.sync_copy(nb_hbm.at[pl.ds(base * M, PAIRS)], idx_v)

        cps = [
            pltpu.async_copy(xt_hbm.at[idx_v], xr, gsem),
            pltpu.async_copy(yt_hbm.at[idx_v], yr, gsem),
            pltpu.async_copy(zt_hbm.at[idx_v], zr, gsem),
            pltpu.async_copy(qt_hbm.at[idx_v], qr, gsem),
        ]
        for cp in cps:
            cp.wait()

        off = b * CHUNK
        xi = ox[pl.ds(off, L)]
        yi = oy[pl.ds(off, L)]
        zi = oz[pl.ds(off, L)]
        qi = oq[pl.ds(off, L)]
        ids = iota + base

        def slot_body(m, acc):
            ridx = iota * M + m
            xj = plsc.load_gather(xr, [ridx])
            yj = plsc.load_gather(yr, [ridx])
            zj = plsc.load_gather(zr, [ridx])
            qj = plsc.load_gather(qr, [ridx])
            nbj = plsc.load_gather(idx_v, [ridx])
            dx = xi - xj
            dy = yi - yj
            dz = zi - zj
            s = dx * dx + dy * dy + dz * dz + 1e-12
            valid = (nbj != ids) & (s > 1e-6)
            s_safe = jnp.where(valid, s, 1.0)
            ibits = lax.bitcast_convert_type(s_safe, jnp.int32)
            magic = jnp.full((L,), 0x5F3759DF, jnp.int32)
            y = lax.bitcast_convert_type(magic - (ibits >> 1), jnp.float32)
            hs = 0.5 * s_safe
            for _ in range(3):
                y = y * (1.5 - hs * y * y)
            x2 = s_safe * inv_rc2
            x2c = jnp.minimum(x2, 0.999999)
            u = 1.0 - 1.0 / (1.0 - x2c)
            bump = jnp.where(x2 < 1.0, jnp.exp(u), 0.0)
            e = (1.0 - bump) * (qi * qj) * y
            return acc + jnp.where(valid, e, 0.0)

        acc = lax.fori_loop(0, M, slot_body, zf, unroll=4)
        mol16 = omol[pl.ds(off, L)]
        plsc.addupdate_scatter(molacc, [iota * NMOL + mol16], acc)
        return 0

    lax.fori_loop(0, NBLK, block_body, 0)

    factor = jnp.float32(FACTOR)

    def comb_body(g, _):
        def row_body(r, v):
            return v + molacc[pl.ds(r * NMOL + g * L, L)]
        v = lax.fori_loop(0, L, row_body, zf, unroll=4)
        res_v[pl.ds(g * L, L)] = v * factor
        return 0

    lax.fori_loop(0, NMOL // L, comb_body, 0)
    pltpu.sync_copy(res_v, out_hbm.at[wid])


def kernel(coord, charges, nbmat_lr, mol_idx):
    pad = NPAD - N
    xt = jnp.pad(coord[:, 0], (0, pad))
    yt = jnp.pad(coord[:, 1], (0, pad))
    zt = jnp.pad(coord[:, 2], (0, pad))
    qt = jnp.pad(charges, (0, pad))
    nb = jnp.pad(nbmat_lr.astype(jnp.int32), ((0, pad), (0, 0))).reshape(-1)
    mol = jnp.pad(mol_idx.astype(jnp.int32), (0, pad))
    partials = _lr_coulomb_sc(xt, yt, zt, qt, nb, mol)
    return jnp.sum(partials, axis=0)


# speedup vs baseline: 59.8656x; 1.5361x over previous
import functools

import jax
import jax.numpy as jnp
from jax import lax
from jax.experimental import pallas as pl
from jax.experimental.pallas import tpu as pltpu
from jax.experimental.pallas import tpu_sc as plsc

N = 100000
M = 64
NMOL = 256
RC = 4.6
FACTOR = 13.605693122994 * 0.529177210903

NC = 2
NS = 16
NW = NC * NS
L = 16

SUP = 32
PER_TILE = 3136
NPAD = NW * PER_TILE
NSUP = PER_TILE // SUP
HMAX = NSUP // 2 - 1
SPAIRS = SUP * M

_mesh = plsc.VectorSubcoreMesh(
    core_axis_name="c", subcore_axis_name="s", num_cores=NC, num_subcores=NS
)


@functools.partial(
    pl.kernel,
    out_type=jax.ShapeDtypeStruct((NW, NMOL), jnp.float32),
    mesh=_mesh,
    compiler_params=pltpu.CompilerParams(needs_layout_passes=False),
    scratch_types=[
        pltpu.VMEM((SPAIRS,), jnp.int32),
        pltpu.VMEM((SPAIRS,), jnp.int32),
        pltpu.VMEM((SPAIRS,), jnp.float32),
        pltpu.VMEM((SPAIRS,), jnp.float32),
        pltpu.VMEM((SPAIRS,), jnp.float32),
        pltpu.VMEM((SPAIRS,), jnp.float32),
        pltpu.VMEM((SPAIRS,), jnp.float32),
        pltpu.VMEM((SPAIRS,), jnp.float32),
        pltpu.VMEM((SPAIRS,), jnp.float32),
        pltpu.VMEM((SPAIRS,), jnp.float32),
        pltpu.VMEM((PER_TILE,), jnp.float32),
        pltpu.VMEM((PER_TILE,), jnp.float32),
        pltpu.VMEM((PER_TILE,), jnp.float32),
        pltpu.VMEM((PER_TILE,), jnp.float32),
        pltpu.VMEM((PER_TILE,), jnp.int32),
        pltpu.VMEM((L * NMOL,), jnp.float32),
        pltpu.VMEM((NMOL,), jnp.float32),
        pltpu.SemaphoreType.DMA,
        pltpu.SemaphoreType.DMA,
        pltpu.SemaphoreType.DMA,
        pltpu.SemaphoreType.DMA,
        pltpu.SemaphoreType.DMA,
    ],
)
def _lr_coulomb_sc(xt_hbm, yt_hbm, zt_hbm, qt_hbm, nb_hbm, mol_hbm, out_hbm,
                   idx0, idx1, xr0, yr0, zr0, qr0, xr1, yr1, zr1, qr1,
                   ox, oy, oz, oq, omol, molacc, res_v,
                   gsem0, gsem1, isem0, isem1, ssem):
    wid = lax.axis_index("s") * NC + lax.axis_index("c")
    start = wid * PER_TILE

    idx_b = (idx0, idx1)
    rows_b = ((xr0, yr0, zr0, qr0), (xr1, yr1, zr1, qr1))
    gsem_b = (gsem0, gsem1)
    isem_b = (isem0, isem1)
    tabs = (xt_hbm, yt_hbm, zt_hbm, qt_hbm)

    iota = lax.broadcasted_iota(jnp.int32, (L,), 0)
    zf = jnp.zeros((L,), jnp.float32)
    inv_rc2 = jnp.float32(1.0 / (RC * RC))

    own_cps = [
        pltpu.async_copy(xt_hbm.at[pl.ds(start, PER_TILE)], ox, ssem),
        pltpu.async_copy(yt_hbm.at[pl.ds(start, PER_TILE)], oy, ssem),
        pltpu.async_copy(zt_hbm.at[pl.ds(start, PER_TILE)], oz, ssem),
        pltpu.async_copy(qt_hbm.at[pl.ds(start, PER_TILE)], oq, ssem),
        pltpu.async_copy(mol_hbm.at[pl.ds(start, PER_TILE)], omol, ssem),
    ]

    def zero_body(i, _):
        molacc[pl.ds(i * L, L)] = zf
        return 0
    lax.fori_loop(0, (L * NMOL) // L, zero_body, 0)
    for cp in own_cps:
        cp.wait()

    def issue_idx(g, p):
        pltpu.async_copy(
            nb_hbm.at[pl.ds((start + g * SUP) * M, SPAIRS)], idx_b[p],
            isem_b[p])

    def drain_idx(p):
        pltpu.make_async_copy(
            nb_hbm.at[pl.ds(0, SPAIRS)], idx_b[p], isem_b[p]).wait()

    def fire_gathers(g, p):
        for f in range(4):
            pltpu.async_copy(tabs[f].at[idx_b[p]], rows_b[p][f], gsem_b[p])

    def drain_gathers(p):
        for f in range(4):
            pltpu.make_async_copy(
                tabs[f].at[pl.ds(0, SPAIRS)], rows_b[p][f], gsem_b[p]).wait()

    def compute(g, p):
        idx_v = idx_b[p]
        xr, yr, zr, qr = rows_b[p]
        base0 = start + g * SUP
        for sub in range(SUP // L):
            off = g * SUP + sub * L
            xi = ox[pl.ds(off, L)]
            yi = oy[pl.ds(off, L)]
            zi = oz[pl.ds(off, L)]
            qi = oq[pl.ds(off, L)]
            ids = iota + (base0 + sub * L)
            pbase = sub * L * M

            def slot_body(m, acc):
                ridx = iota * M + (pbase + m)
                xj = plsc.load_gather(xr, [ridx])
                yj = plsc.load_gather(yr, [ridx])
                zj = plsc.load_gather(zr, [ridx])
                qj = plsc.load_gather(qr, [ridx])
                nbj = plsc.load_gather(idx_v, [ridx])
                dx = xi - xj
                dy = yi - yj
                dz = zi - zj
                s = dx * dx + dy * dy + dz * dz + 1e-12
                valid = (nbj != ids) & (s > 1e-6)
                s_safe = jnp.where(valid, s, 1.0)
                ibits = lax.bitcast_convert_type(s_safe, jnp.int32)
                magic = jnp.full((L,), 0x5F3759DF, jnp.int32)
                y = lax.bitcast_convert_type(magic - (ibits >> 1), jnp.float32)
                hs = 0.5 * s_safe
                for _ in range(3):
                    y = y * (1.5 - hs * y * y)
                x2 = s_safe * inv_rc2
                x2c = jnp.minimum(x2, 0.999999)
                u = 1.0 - 1.0 / (1.0 - x2c)
                bump = jnp.where(x2 < 1.0, jnp.exp(u), 0.0)
                e = (1.0 - bump) * (qi * qj) * y
                return acc + jnp.where(valid, e, 0.0)

            acc = lax.fori_loop(0, M, slot_body, zf, unroll=4)
            mol16 = omol[pl.ds(off, L)]
            plsc.addupdate_scatter(molacc, [iota * NMOL + mol16], acc)

    issue_idx(0, 0)
    drain_idx(0)
    fire_gathers(0, 0)
    issue_idx(1, 1)

    def pair_body(h, _):
        g = 2 * h
        drain_idx(1)
        fire_gathers(g + 1, 1)
        drain_gathers(0)
        compute(g, 0)

        @pl.when(h < HMAX)
        def _():
            issue_idx(g + 2, 0)

        @pl.when(h < HMAX)
        def _():
            drain_idx(0)
            fire_gathers(g + 2, 0)
        drain_gathers(1)
        compute(g + 1, 1)

        @pl.when(h < HMAX)
        def _():
            issue_idx(g + 3, 1)
        return 0

    lax.fori_loop(0, NSUP // 2, pair_body, 0)

    factor = jnp.float32(FACTOR)

    def comb_body(gg, _):
        def row_body(r, v):
            return v + molacc[pl.ds(r * NMOL + gg * L, L)]
        v = lax.fori_loop(0, L, row_body, zf, unroll=4)
        res_v[pl.ds(gg * L, L)] = v * factor
        return 0

    lax.fori_loop(0, NMOL // L, comb_body, 0)
    pltpu.sync_copy(res_v, out_hbm.at[wid])


def kernel(coord, charges, nbmat_lr, mol_idx):
    pad = NPAD - N
    xt = jnp.pad(coord[:, 0], (0, pad))
    yt = jnp.pad(coord[:, 1], (0, pad))
    zt = jnp.pad(coord[:, 2], (0, pad))
    qt = jnp.pad(charges, (0, pad))
    nb = jnp.pad(nbmat_lr.astype(jnp.int32), ((0, pad), (0, 0))).reshape(-1)
    mol = jnp.pad(mol_idx.astype(jnp.int32), (0, pad))
    partials = _lr_coulomb_sc(xt, yt, zt, qt, nb, mol)
    return jnp.sum(partials, axis=0)
